# Initial kernel scaffold; baseline (speedup 1.0000x reference)
import functools

import jax
import jax.numpy as jnp
from jax import lax
from jax.experimental import pallas as pl
from jax.experimental.pallas import tpu as pltpu
from jax.experimental.pallas import tpu_sc as plsc

N_NODES = 10000
N_EDGES = 320000
NP = 10240

NC = 2
NS = 16
NW = NC * NS
EW = N_EDGES // NW
CW = 80
CH = EW // CW
RPT = N_NODES // NS
HPT = NP // NS

_SC_MESH = dict(core_axis_name="c", subcore_axis_name="s")


def _degree_kernel(srcr, dstr, degp, src_v, dst_v, ones_v, zb_v, hsrc, hdst):
    c = lax.axis_index("c")
    s = lax.axis_index("s")
    w = c * NS + s

    def _fill(i, _):
        zb_v[pl.ds(i * 16, 16)] = jnp.zeros((16,), jnp.float32)
        return 0
    lax.fori_loop(0, HPT // 16, _fill, 0)
    for i in range(CW // 16):
        ones_v[pl.ds(i * 16, 16)] = jnp.ones((16,), jnp.float32)

    pltpu.sync_copy(zb_v, hsrc.at[pl.ds(s * HPT, HPT)])
    pltpu.sync_copy(zb_v, hdst.at[pl.ds(s * HPT, HPT)])

    pltpu.sync_copy(srcr.at[w], src_v)
    pltpu.sync_copy(dstr.at[w], dst_v)
    plsc.subcore_barrier()

    def _body(j, _):
        pltpu.sync_copy(ones_v, hsrc.at[src_v.at[j]], add=True)
        pltpu.sync_copy(ones_v, hdst.at[dst_v.at[j]], add=True)
        return 0
    lax.fori_loop(0, CH, _body, 0)

    plsc.subcore_barrier()
    pltpu.sync_copy(hsrc.at[pl.ds(s * HPT, HPT)], degp.at[c, 0, pl.ds(s * HPT, HPT)])
    pltpu.sync_copy(hdst.at[pl.ds(s * HPT, HPT)], degp.at[c, 1, pl.ds(s * HPT, HPT)])


_degree_call = functools.partial(
    pl.kernel,
    out_type=jax.ShapeDtypeStruct((NC, 2, NP), jnp.float32),
    mesh=plsc.VectorSubcoreMesh(**_SC_MESH),
    scratch_types=[
        pltpu.VMEM((CH, CW), jnp.int32),
        pltpu.VMEM((CH, CW), jnp.int32),
        pltpu.VMEM((CW,), jnp.float32),
        pltpu.VMEM((HPT,), jnp.float32),
        pltpu.VMEM_SHARED((NP,), jnp.float32),
        pltpu.VMEM_SHARED((NP,), jnp.float32),
    ],
)(_degree_kernel)


def _prop_kernel(feats, srcr, dstr, part, src_v, dst_v, rows0, rows1, acc,
                 sem0, sem1, *, F):
    c = lax.axis_index("c")
    s = lax.axis_index("s")
    w = c * NS + s

    def _fill(r, _):
        for i in range(F // 16):
            rows0[r, pl.ds(i * 16, 16)] = jnp.zeros((16,), jnp.float32)
        return 0
    lax.fori_loop(0, CW, _fill, 0)
    base = s * RPT
    for k in range(RPT // CW):
        pltpu.sync_copy(rows0, acc.at[pl.ds(base + k * CW, CW)])
    rem = RPT - (RPT // CW) * CW
    pltpu.sync_copy(rows0.at[pl.ds(0, rem)], acc.at[pl.ds(base + (RPT // CW) * CW, rem)])

    pltpu.sync_copy(srcr.at[w], src_v)
    pltpu.sync_copy(dstr.at[w], dst_v)
    plsc.subcore_barrier()

    def _wait(buf):
        pltpu.make_async_copy(feats.at[pl.ds(0, CW)], buf, sem0 if buf is rows0 else sem1).wait()

    pltpu.async_copy(feats.at[src_v.at[0]], rows0, sem0)

    def _body(i, _):
        j0 = 2 * i
        _wait(rows0)
        pltpu.async_copy(feats.at[src_v.at[j0 + 1]], rows1, sem1)
        pltpu.sync_copy(rows0, acc.at[dst_v.at[j0]], add=True)
        _wait(rows1)
        pltpu.async_copy(feats.at[src_v.at[j0 + 2]], rows0, sem0)
        pltpu.sync_copy(rows1, acc.at[dst_v.at[j0 + 1]], add=True)
        return 0
    lax.fori_loop(0, (CH - 1) // 2, _body, 0)

    _wait(rows0)
    pltpu.sync_copy(rows0, acc.at[dst_v.at[CH - 1]], add=True)

    plsc.subcore_barrier()
    pltpu.sync_copy(acc.at[pl.ds(base, RPT)], part.at[c, pl.ds(base, RPT)])


def _make_prop(F):
    return functools.partial(
        pl.kernel,
        out_type=jax.ShapeDtypeStruct((NC, N_NODES, F), jnp.float32),
        mesh=plsc.VectorSubcoreMesh(**_SC_MESH),
        scratch_types=[
            pltpu.VMEM((CH, CW), jnp.int32),
            pltpu.VMEM((CH, CW), jnp.int32),
            pltpu.VMEM((CW, F), jnp.float32),
            pltpu.VMEM((CW, F), jnp.float32),
            pltpu.VMEM_SHARED((N_NODES, F), jnp.float32),
            pltpu.SemaphoreType.DMA,
            pltpu.SemaphoreType.DMA,
        ],
    )(functools.partial(_prop_kernel, F=F))


_prop128_call = _make_prop(128)
_prop64_call = _make_prop(64)


_R = 2048
_GRID = (NP // _R,)


def _norms(degp_ref):
    deg_out = degp_ref[0, 0, :] + degp_ref[1, 0, :]
    deg_in = degp_ref[0, 1, :] + degp_ref[1, 1, :]
    no = lax.rsqrt(jnp.maximum(deg_out, 1.0))
    ni = lax.rsqrt(jnp.maximum(deg_in, 1.0))
    return no, ni


def _scale_kernel(x_ref, degp_ref, out_ref):
    no, _ = _norms(degp_ref)
    out_ref[...] = x_ref[...] * no[:, None]


def _mid_kernel(p_ref, degp_ref, w1_ref, b1_ref, w2_ref, out_ref):
    no, ni = _norms(degp_ref)
    agg = (p_ref[0] + p_ref[1]) * ni[:, None]
    h = jnp.maximum(jnp.dot(agg, w1_ref[...], preferred_element_type=jnp.float32)
                    + b1_ref[...], 0.0)
    out_ref[...] = jnp.dot(h, w2_ref[...], preferred_element_type=jnp.float32) * no[:, None]


def _final_kernel(q_ref, degp_ref, b2_ref, out_ref):
    _, ni = _norms(degp_ref)
    out_ref[...] = (q_ref[0] + q_ref[1]) * ni[:, None] + b2_ref[...]


_DEGP_SPEC = pl.BlockSpec((NC, 2, _R), lambda i: (0, 0, i))


def _tc_scale(x, degp):
    return pl.pallas_call(
        _scale_kernel,
        grid=_GRID,
        in_specs=[pl.BlockSpec((_R, 128), lambda i: (i, 0)), _DEGP_SPEC],
        out_specs=pl.BlockSpec((_R, 128), lambda i: (i, 0)),
        out_shape=jax.ShapeDtypeStruct((N_NODES, 128), jnp.float32),
    )(x, degp)


def _tc_mid(p, degp, W1, b1, W2):
    return pl.pallas_call(
        _mid_kernel,
        grid=_GRID,
        in_specs=[
            pl.BlockSpec((NC, _R, 128), lambda i: (0, i, 0)),
            _DEGP_SPEC,
            pl.BlockSpec((128, 128), lambda i: (0, 0)),
            pl.BlockSpec((1, 128), lambda i: (0, 0)),
            pl.BlockSpec((128, 64), lambda i: (0, 0)),
        ],
        out_specs=pl.BlockSpec((_R, 64), lambda i: (i, 0)),
        out_shape=jax.ShapeDtypeStruct((N_NODES, 64), jnp.float32),
    )(p, degp, W1, b1, W2)


def _tc_final(q, degp, b2):
    return pl.pallas_call(
        _final_kernel,
        grid=_GRID,
        in_specs=[
            pl.BlockSpec((NC, _R, 64), lambda i: (0, i, 0)),
            _DEGP_SPEC,
            pl.BlockSpec((1, 64), lambda i: (0, 0)),
        ],
        out_specs=pl.BlockSpec((_R, 64), lambda i: (i, 0)),
        out_shape=jax.ShapeDtypeStruct((N_NODES, 64), jnp.float32),
    )(q, degp, b2)


def kernel(in_feat, edge_index, W1, b1, W2, b2):
    er = edge_index.astype(jnp.int32)
    srcr = er[0].reshape(NW, CH, CW)
    dstr = er[1].reshape(NW, CH, CW)

    degp = _degree_call(srcr, dstr)
    xn1 = _tc_scale(in_feat, degp)
    p1 = _prop128_call(xn1, srcr, dstr)
    g = _tc_mid(p1, degp, W1.astype(jnp.float32), b1.reshape(1, 128),
                W2.astype(jnp.float32))
    p2 = _prop64_call(g, srcr, dstr)
    return _tc_final(p2, degp, b2.reshape(1, 64))


# speedup vs baseline: 7.5336x; 7.5336x over previous
import functools

import jax
import jax.numpy as jnp
from jax import lax
from jax.experimental import pallas as pl
from jax.experimental.pallas import tpu as pltpu
from jax.experimental.pallas import tpu_sc as plsc

N_NODES = 10000
N_EDGES = 320000
NP = 10240

NC = 2
NS = 16
NW = NC * NS
EW = N_EDGES // NW
CW = 80
CH = EW // CW
EP = N_EDGES // NS
PW = 80
PH = EP // PW
RPT = NP // NS
HPT = NP // NS

_SC_MESH = dict(core_axis_name="c", subcore_axis_name="s",
                num_cores=NC, num_subcores=NS)


def _degree_kernel(srcr, dstr, degp, src_v, dst_v, ones_v, zb_v, hsrc, hdst):
    c = lax.axis_index("c")
    s = lax.axis_index("s")
    w = c * NS + s

    def _fill(i, _):
        zb_v[pl.ds(i * 16, 16)] = jnp.zeros((16,), jnp.float32)
        return 0
    lax.fori_loop(0, HPT // 16, _fill, 0)
    for i in range(CW // 16):
        ones_v[pl.ds(i * 16, 16)] = jnp.ones((16,), jnp.float32)

    pltpu.sync_copy(zb_v, hsrc.at[pl.ds(s * HPT, HPT)])
    pltpu.sync_copy(zb_v, hdst.at[pl.ds(s * HPT, HPT)])

    pltpu.sync_copy(srcr.at[w], src_v)
    pltpu.sync_copy(dstr.at[w], dst_v)
    plsc.subcore_barrier()

    def _body(j, _):
        pltpu.sync_copy(ones_v, hsrc.at[src_v.at[j]], add=True)
        pltpu.sync_copy(ones_v, hdst.at[dst_v.at[j]], add=True)
        return 0
    lax.fori_loop(0, CH, _body, 0)

    plsc.subcore_barrier()
    pltpu.sync_copy(hsrc.at[pl.ds(s * HPT, HPT)],
                    degp.at[pl.ds((c * 2 + 0) * NP + s * HPT, HPT)])
    pltpu.sync_copy(hdst.at[pl.ds(s * HPT, HPT)],
                    degp.at[pl.ds((c * 2 + 1) * NP + s * HPT, HPT)])


_degree_call = functools.partial(
    pl.kernel,
    out_type=jax.ShapeDtypeStruct((NC * 2 * NP,), jnp.float32),
    mesh=plsc.VectorSubcoreMesh(**_SC_MESH),
    scratch_types=[
        pltpu.VMEM((CH, CW), jnp.int32),
        pltpu.VMEM((CH, CW), jnp.int32),
        pltpu.VMEM((CW,), jnp.float32),
        pltpu.VMEM((HPT,), jnp.float32),
        pltpu.VMEM_SHARED((NP,), jnp.float32),
        pltpu.VMEM_SHARED((NP,), jnp.float32),
    ],
)(_degree_kernel)


def _prop_kernel(feats, srcx, dstr, part, src_v, dst_v, rows0, rows1, acc,
                 sem0, sem1, *, FH):
    c = lax.axis_index("c")
    s = lax.axis_index("s")

    def _fill(r, _):
        for i in range(FH // 16):
            rows0[r, pl.ds(i * 16, 16)] = jnp.zeros((16,), jnp.float32)
        return 0
    lax.fori_loop(0, PW, _fill, 0)
    base = s * RPT
    for k in range(RPT // 80):
        pltpu.sync_copy(rows0.at[pl.ds(0, 80)], acc.at[pl.ds(base + k * 80, 80)])

    pltpu.sync_copy(srcx.at[c, s], src_v)
    pltpu.sync_copy(dstr.at[s], dst_v)
    plsc.subcore_barrier()

    def _wait(buf, sem):
        pltpu.make_async_copy(feats.at[pl.ds(0, PW)], buf, sem).wait()

    pltpu.async_copy(feats.at[src_v.at[0]], rows0, sem0)

    def _body(i, _):
        j0 = 2 * i
        _wait(rows0, sem0)
        pltpu.async_copy(feats.at[src_v.at[j0 + 1]], rows1, sem1)
        pltpu.sync_copy(rows0, acc.at[dst_v.at[j0]], add=True)
        _wait(rows1, sem1)
        pltpu.async_copy(feats.at[src_v.at[j0 + 2]], rows0, sem0)
        pltpu.sync_copy(rows1, acc.at[dst_v.at[j0 + 1]], add=True)
        return 0
    lax.fori_loop(0, PH // 2 - 1, _body, 0)

    _wait(rows0, sem0)
    pltpu.async_copy(feats.at[src_v.at[PH - 1]], rows1, sem1)
    pltpu.sync_copy(rows0, acc.at[dst_v.at[PH - 2]], add=True)
    _wait(rows1, sem1)
    pltpu.sync_copy(rows1, acc.at[dst_v.at[PH - 1]], add=True)

    plsc.subcore_barrier()
    pltpu.sync_copy(acc.at[pl.ds(base, RPT)], part.at[c, pl.ds(base, RPT)])


def _make_prop(FH):
    return functools.partial(
        pl.kernel,
        out_type=jax.ShapeDtypeStruct((NC, NP, FH), jnp.float32),
        mesh=plsc.VectorSubcoreMesh(**_SC_MESH),
        scratch_types=[
            pltpu.VMEM((PH, PW), jnp.int32),
            pltpu.VMEM((PH, PW), jnp.int32),
            pltpu.VMEM((PW, FH), jnp.float32),
            pltpu.VMEM((PW, FH), jnp.float32),
            pltpu.VMEM_SHARED((NP, FH), jnp.float32),
            pltpu.SemaphoreType.DMA,
            pltpu.SemaphoreType.DMA,
        ],
        compiler_params=pltpu.CompilerParams(use_tc_tiling_on_sc=False),
    )(functools.partial(_prop_kernel, FH=FH))


_prop64_call = _make_prop(64)
_prop32_call = _make_prop(32)


_R = 2048
_GRID = (NP // _R,)


def _norms(degp_ref):
    deg_out = degp_ref[0, 0, :] + degp_ref[1, 0, :]
    deg_in = degp_ref[0, 1, :] + degp_ref[1, 1, :]
    no = lax.rsqrt(jnp.maximum(deg_out, 1.0))
    ni = lax.rsqrt(jnp.maximum(deg_in, 1.0))
    return no, ni


def _scale_kernel(x_ref, degp_ref, out_ref):
    no, _ = _norms(degp_ref)
    xn = x_ref[...] * no[:, None]
    out_ref[0] = xn[:, :64]
    out_ref[1] = xn[:, 64:]


def _mid_kernel(p_ref, degp_ref, w1_ref, b1_ref, w2_ref, out_ref):
    no, ni = _norms(degp_ref)
    agg = jnp.concatenate([p_ref[0], p_ref[1]], axis=1) * ni[:, None]
    h = jnp.maximum(jnp.dot(agg, w1_ref[...], preferred_element_type=jnp.float32)
                    + b1_ref[...], 0.0)
    g = jnp.dot(h, w2_ref[...], preferred_element_type=jnp.float32) * no[:, None]
    out_ref[0] = g[:, :32]
    out_ref[1] = g[:, 32:]


def _final_kernel(q_ref, degp_ref, b2_ref, out_ref):
    _, ni = _norms(degp_ref)
    out_ref[...] = (jnp.concatenate([q_ref[0], q_ref[1]], axis=1)
                    * ni[:, None] + b2_ref[...])


_DEGP_SPEC = pl.BlockSpec((NC, 2, _R), lambda i: (0, 0, i))


def _tc_scale(x, degp):
    return pl.pallas_call(
        _scale_kernel,
        grid=_GRID,
        in_specs=[pl.BlockSpec((_R, 128), lambda i: (i, 0)), _DEGP_SPEC],
        out_specs=pl.BlockSpec((NC, _R, 64), lambda i: (0, i, 0)),
        out_shape=jax.ShapeDtypeStruct((NC, NP, 64), jnp.float32),
    )(x, degp)


def _tc_mid(p, degp, W1, b1, W2):
    return pl.pallas_call(
        _mid_kernel,
        grid=_GRID,
        in_specs=[
            pl.BlockSpec((NC, _R, 64), lambda i: (0, i, 0)),
            _DEGP_SPEC,
            pl.BlockSpec((128, 128), lambda i: (0, 0)),
            pl.BlockSpec((1, 128), lambda i: (0, 0)),
            pl.BlockSpec((128, 64), lambda i: (0, 0)),
        ],
        out_specs=pl.BlockSpec((NC, _R, 32), lambda i: (0, i, 0)),
        out_shape=jax.ShapeDtypeStruct((NC, NP, 32), jnp.float32),
    )(p, degp, W1, b1, W2)


def _tc_final(q, degp, b2):
    return pl.pallas_call(
        _final_kernel,
        grid=_GRID,
        in_specs=[
            pl.BlockSpec((NC, _R, 32), lambda i: (0, i, 0)),
            _DEGP_SPEC,
            pl.BlockSpec((1, 64), lambda i: (0, 0)),
        ],
        out_specs=pl.BlockSpec((_R, 64), lambda i: (i, 0)),
        out_shape=jax.ShapeDtypeStruct((N_NODES, 64), jnp.float32),
    )(q, degp, b2)


def kernel(in_feat, edge_index, W1, b1, W2, b2):
    er = edge_index.astype(jnp.int32)
    src = er[0]
    dst = er[1]
    srcr = src.reshape(NW, CH, CW)
    dstr = dst.reshape(NW, CH, CW)
    srcp = src.reshape(NS, PH, PW)
    srcx = jnp.stack([srcp, srcp + NP])
    dstp = dst.reshape(NS, PH, PW)

    degp = _degree_call(srcr, dstr).reshape(NC, 2, NP)
    xn1 = _tc_scale(in_feat, degp).reshape(NC * NP, 64)
    p1 = _prop64_call(xn1, srcx, dstp)
    g = _tc_mid(p1, degp, W1.astype(jnp.float32), b1.reshape(1, 128),
                W2.astype(jnp.float32)).reshape(NC * NP, 32)
    p2 = _prop32_call(g, srcx, dstp)
    return _tc_final(p2, degp, b2.reshape(1, 64))


# speedup vs baseline: 10.6501x; 1.4137x over previous
import functools

import jax
import jax.numpy as jnp
from jax import lax
from jax.experimental import pallas as pl
from jax.experimental.pallas import tpu as pltpu
from jax.experimental.pallas import tpu_sc as plsc

N_NODES = 10000
N_EDGES = 320000
NP = 10240

NC = 2
NS = 16
NW = NC * NS
PW = 128
T1 = 157
T2 = 79
RPT = NP // NS
HPT = NP // NS

_SC_MESH = dict(core_axis_name="c", subcore_axis_name="s",
                num_cores=NC, num_subcores=NS)
_SC_PARAMS = pltpu.CompilerParams(use_tc_tiling_on_sc=False)


def _degree_kernel(srcx, dstx, degp, src_v, dst_v, ones_v, zb_v, hsrc, hdst):
    c = lax.axis_index("c")
    s = lax.axis_index("s")

    def _fill(i, _):
        zb_v[pl.ds(i * 16, 16)] = jnp.zeros((16,), jnp.float32)
        return 0
    lax.fori_loop(0, HPT // 16, _fill, 0)
    for i in range(PW // 16):
        ones_v[pl.ds(i * 16, 16)] = jnp.ones((16,), jnp.float32)

    pltpu.sync_copy(zb_v, hsrc.at[pl.ds(s * HPT, HPT)])
    pltpu.sync_copy(zb_v, hdst.at[pl.ds(s * HPT, HPT)])

    pltpu.sync_copy(srcx.at[c, s], src_v)
    pltpu.sync_copy(dstx.at[c, s], dst_v)
    plsc.subcore_barrier()

    def _body(j, _):
        pltpu.sync_copy(ones_v, hsrc.at[src_v.at[j]], add=True)
        pltpu.sync_copy(ones_v, hdst.at[dst_v.at[j]], add=True)
        return 0
    lax.fori_loop(0, T2, _body, 0)

    plsc.subcore_barrier()
    pltpu.sync_copy(hsrc.at[pl.ds(s * HPT, HPT)],
                    degp.at[pl.ds((c * 2 + 0) * NP + s * HPT, HPT)])
    pltpu.sync_copy(hdst.at[pl.ds(s * HPT, HPT)],
                    degp.at[pl.ds((c * 2 + 1) * NP + s * HPT, HPT)])


_degree_call = functools.partial(
    pl.kernel,
    out_type=jax.ShapeDtypeStruct((NC * 2 * NP,), jnp.float32),
    mesh=plsc.VectorSubcoreMesh(**_SC_MESH),
    scratch_types=[
        pltpu.VMEM((T2, PW), jnp.int32),
        pltpu.VMEM((T2, PW), jnp.int32),
        pltpu.VMEM((PW,), jnp.float32),
        pltpu.VMEM((HPT,), jnp.float32),
        pltpu.VMEM_SHARED((NP,), jnp.float32),
        pltpu.VMEM_SHARED((NP,), jnp.float32),
    ],
    compiler_params=_SC_PARAMS,
)(_degree_kernel)


def _prop_kernel(feats, srcx, dstx, part, src_v, dst_v, rows0, rows1, acc,
                 sem0, sem1, *, T):
    c = lax.axis_index("c")
    s = lax.axis_index("s")

    def _fill(r, _):
        for i in range(64 // 16):
            rows0[r, pl.ds(i * 16, 16)] = jnp.zeros((16,), jnp.float32)
        return 0
    lax.fori_loop(0, PW, _fill, 0)
    base = s * RPT
    for k in range(RPT // PW):
        pltpu.sync_copy(rows0, acc.at[pl.ds(base + k * PW, PW)])

    pltpu.sync_copy(srcx.at[c, s], src_v)
    pltpu.sync_copy(dstx.at[c, s], dst_v)
    plsc.subcore_barrier()

    def _wait(buf, sem):
        pltpu.make_async_copy(feats.at[pl.ds(0, PW)], buf, sem).wait()

    pltpu.async_copy(feats.at[src_v.at[0]], rows0, sem0)

    def _body(i, _):
        j0 = 2 * i
        _wait(rows0, sem0)
        pltpu.async_copy(feats.at[src_v.at[j0 + 1]], rows1, sem1)
        pltpu.sync_copy(rows0, acc.at[dst_v.at[j0]], add=True)
        _wait(rows1, sem1)
        pltpu.async_copy(feats.at[src_v.at[j0 + 2]], rows0, sem0)
        pltpu.sync_copy(rows1, acc.at[dst_v.at[j0 + 1]], add=True)
        return 0
    lax.fori_loop(0, (T - 1) // 2, _body, 0)

    _wait(rows0, sem0)
    pltpu.sync_copy(rows0, acc.at[dst_v.at[T - 1]], add=True)

    plsc.subcore_barrier()
    pltpu.sync_copy(acc.at[pl.ds(base, RPT)], part.at[c, pl.ds(base, RPT)])


def _make_prop(T):
    return functools.partial(
        pl.kernel,
        out_type=jax.ShapeDtypeStruct((NC, NP, 64), jnp.float32),
        mesh=plsc.VectorSubcoreMesh(**_SC_MESH),
        scratch_types=[
            pltpu.VMEM((T, PW), jnp.int32),
            pltpu.VMEM((T, PW), jnp.int32),
            pltpu.VMEM((PW, 64), jnp.float32),
            pltpu.VMEM((PW, 64), jnp.float32),
            pltpu.VMEM_SHARED((NP, 64), jnp.float32),
            pltpu.SemaphoreType.DMA,
            pltpu.SemaphoreType.DMA,
        ],
        compiler_params=_SC_PARAMS,
    )(functools.partial(_prop_kernel, T=T))


_prop1_call = _make_prop(T1)
_prop2_call = _make_prop(T2)


_R = 2048
_GRID = (NP // _R,)


def _norms(degp_ref):
    deg_out = degp_ref[0, 0, :] + degp_ref[1, 0, :]
    deg_in = degp_ref[0, 1, :] + degp_ref[1, 1, :]
    no = lax.rsqrt(jnp.maximum(deg_out, 1.0))
    ni = lax.rsqrt(jnp.maximum(deg_in, 1.0))
    return no, ni


def _scale_kernel(x_ref, degp_ref, out_ref):
    no, _ = _norms(degp_ref)
    xn = x_ref[...] * no[:, None]
    out_ref[0] = xn[:, :64]
    out_ref[1] = xn[:, 64:]


def _mid_kernel(p_ref, degp_ref, w1_ref, b1_ref, w2_ref, out_ref):
    no, ni = _norms(degp_ref)
    agg = jnp.concatenate([p_ref[0], p_ref[1]], axis=1) * ni[:, None]
    h = jnp.maximum(jnp.dot(agg, w1_ref[...], preferred_element_type=jnp.float32)
                    + b1_ref[...], 0.0)
    out_ref[...] = jnp.dot(h, w2_ref[...],
                           preferred_element_type=jnp.float32) * no[:, None]


def _final_kernel(q_ref, degp_ref, b2_ref, out_ref):
    _, ni = _norms(degp_ref)
    out_ref[...] = (q_ref[0] + q_ref[1]) * ni[:, None] + b2_ref[...]


_DEGP_SPEC = pl.BlockSpec((NC, 2, _R), lambda i: (0, 0, i))


def _tc_scale(x, degp):
    return pl.pallas_call(
        _scale_kernel,
        grid=_GRID,
        in_specs=[pl.BlockSpec((_R, 128), lambda i: (i, 0)), _DEGP_SPEC],
        out_specs=pl.BlockSpec((NC, _R, 64), lambda i: (0, i, 0)),
        out_shape=jax.ShapeDtypeStruct((NC, NP, 64), jnp.float32),
    )(x, degp)


def _tc_mid(p, degp, W1, b1, W2):
    return pl.pallas_call(
        _mid_kernel,
        grid=_GRID,
        in_specs=[
            pl.BlockSpec((NC, _R, 64), lambda i: (0, i, 0)),
            _DEGP_SPEC,
            pl.BlockSpec((128, 128), lambda i: (0, 0)),
            pl.BlockSpec((1, 128), lambda i: (0, 0)),
            pl.BlockSpec((128, 64), lambda i: (0, 0)),
        ],
        out_specs=pl.BlockSpec((_R, 64), lambda i: (i, 0)),
        out_shape=jax.ShapeDtypeStruct((NP, 64), jnp.float32),
    )(p, degp, W1, b1, W2)


def _tc_final(q, degp, b2):
    return pl.pallas_call(
        _final_kernel,
        grid=_GRID,
        in_specs=[
            pl.BlockSpec((NC, _R, 64), lambda i: (0, i, 0)),
            _DEGP_SPEC,
            pl.BlockSpec((1, 64), lambda i: (0, 0)),
        ],
        out_specs=pl.BlockSpec((_R, 64), lambda i: (i, 0)),
        out_shape=jax.ShapeDtypeStruct((N_NODES, 64), jnp.float32),
    )(q, degp, b2)


def _pad_chunks(a, rows, t):
    a = a.reshape(rows, -1)
    npad = t * PW - a.shape[1]
    padv = N_NODES + (jnp.arange(npad, dtype=jnp.int32) % (NP - N_NODES))
    pads = jnp.broadcast_to(padv, (rows, npad))
    return jnp.concatenate([a, pads], axis=1).reshape(rows, t, PW)


def kernel(in_feat, edge_index, W1, b1, W2, b2):
    er = edge_index.astype(jnp.int32)
    src = er[0]
    dst = er[1]

    s1 = _pad_chunks(src, NS, T1)
    d1 = _pad_chunks(dst, NS, T1)
    srcx1 = jnp.stack([s1, s1 + NP])
    dstx1 = jnp.stack([d1, d1])

    s2 = _pad_chunks(src, NW, T2).reshape(NC, NS, T2, PW)
    d2 = _pad_chunks(dst, NW, T2).reshape(NC, NS, T2, PW)

    degp = _degree_call(s2, d2).reshape(NC, 2, NP)
    xn1 = _tc_scale(in_feat, degp).reshape(NC * NP, 64)
    p1 = _prop1_call(xn1, srcx1, dstx1)
    g = _tc_mid(p1, degp, W1.astype(jnp.float32), b1.reshape(1, 128),
                W2.astype(jnp.float32))
    p2 = _prop2_call(g, s2, d2)
    return _tc_final(p2, degp, b2.reshape(1, 64))


# speedup vs baseline: 10.8421x; 1.0180x over previous
import functools

import jax
import jax.numpy as jnp
from jax import lax
from jax.experimental import pallas as pl
from jax.experimental.pallas import tpu as pltpu
from jax.experimental.pallas import tpu_sc as plsc

N_NODES = 10000
N_EDGES = 320000
NP = 10240

NC = 2
NS = 16
NW = NC * NS
PW = 128
T = 79
RPT = NP // NS
HPT = NP // NS

_SC_MESH = dict(core_axis_name="c", subcore_axis_name="s",
                num_cores=NC, num_subcores=NS)
_SC_PARAMS = pltpu.CompilerParams(use_tc_tiling_on_sc=False)


def _degree_kernel(srcx, dstx, degp, src_v, dst_v, ones_v, zb_v, hsrc, hdst):
    c = lax.axis_index("c")
    s = lax.axis_index("s")
    w = c * NS + s

    def _fill(i, _):
        zb_v[pl.ds(i * 16, 16)] = jnp.zeros((16,), jnp.float32)
        return 0
    lax.fori_loop(0, HPT // 16, _fill, 0)
    for i in range(PW // 16):
        ones_v[pl.ds(i * 16, 16)] = jnp.ones((16,), jnp.float32)

    pltpu.sync_copy(zb_v, hsrc.at[pl.ds(s * HPT, HPT)])
    pltpu.sync_copy(zb_v, hdst.at[pl.ds(s * HPT, HPT)])

    pltpu.sync_copy(srcx.at[0, w], src_v)
    pltpu.sync_copy(dstx.at[w], dst_v)
    plsc.subcore_barrier()

    def _body(j, _):
        pltpu.sync_copy(ones_v, hsrc.at[src_v.at[j]], add=True)
        pltpu.sync_copy(ones_v, hdst.at[dst_v.at[j]], add=True)
        return 0
    lax.fori_loop(0, T, _body, 0)

    plsc.subcore_barrier()
    pltpu.sync_copy(hsrc.at[pl.ds(s * HPT, HPT)],
                    degp.at[pl.ds((c * 2 + 0) * NP + s * HPT, HPT)])
    pltpu.sync_copy(hdst.at[pl.ds(s * HPT, HPT)],
                    degp.at[pl.ds((c * 2 + 1) * NP + s * HPT, HPT)])


_degree_call = functools.partial(
    pl.kernel,
    out_type=jax.ShapeDtypeStruct((NC * 2 * NP,), jnp.float32),
    mesh=plsc.VectorSubcoreMesh(**_SC_MESH),
    scratch_types=[
        pltpu.VMEM((T, PW), jnp.int32),
        pltpu.VMEM((T, PW), jnp.int32),
        pltpu.VMEM((PW,), jnp.float32),
        pltpu.VMEM((HPT,), jnp.float32),
        pltpu.VMEM_SHARED((NP,), jnp.float32),
        pltpu.VMEM_SHARED((NP,), jnp.float32),
    ],
    compiler_params=_SC_PARAMS,
)(_degree_kernel)


def _prop_kernel(feats, srcx, dstx, part, src_v, dst_v, rows0, rows1, acc,
                 sem0, sem1, *, col_split):
    c = lax.axis_index("c")
    s = lax.axis_index("s")

    def _fill(r, _):
        for i in range(4):
            rows0[r, pl.ds(i * 16, 16)] = jnp.zeros((16,), jnp.float32)
        return 0
    lax.fori_loop(0, PW, _fill, 0)
    base = s * RPT
    for k in range(RPT // PW):
        pltpu.sync_copy(rows0, acc.at[pl.ds(base + k * PW, PW)])

    def _wait(buf, sem):
        pltpu.make_async_copy(feats.at[pl.ds(0, PW)], buf, sem).wait()

    def _run(plane, w):
        pltpu.sync_copy(srcx.at[plane, w], src_v)
        pltpu.sync_copy(dstx.at[w], dst_v)
        plsc.subcore_barrier()
        pltpu.async_copy(feats.at[src_v.at[0]], rows0, sem0)

        def _body(i, _):
            j0 = 2 * i
            _wait(rows0, sem0)
            pltpu.async_copy(feats.at[src_v.at[j0 + 1]], rows1, sem1)
            pltpu.sync_copy(rows0, acc.at[dst_v.at[j0]], add=True)
            _wait(rows1, sem1)
            pltpu.async_copy(feats.at[src_v.at[j0 + 2]], rows0, sem0)
            pltpu.sync_copy(rows1, acc.at[dst_v.at[j0 + 1]], add=True)
            return 0
        lax.fori_loop(0, (T - 1) // 2, _body, 0)

        _wait(rows0, sem0)
        pltpu.sync_copy(rows0, acc.at[dst_v.at[T - 1]], add=True)

    if col_split:
        _run(c, s)
        _run(c, s + NS)
    else:
        _run(0, c * NS + s)

    plsc.subcore_barrier()
    pltpu.sync_copy(acc.at[pl.ds(base, RPT)], part.at[c, pl.ds(base, RPT)])


def _make_prop(col_split):
    return functools.partial(
        pl.kernel,
        out_type=jax.ShapeDtypeStruct((NC, NP, 64), jnp.float32),
        mesh=plsc.VectorSubcoreMesh(**_SC_MESH),
        scratch_types=[
            pltpu.VMEM((T, PW), jnp.int32),
            pltpu.VMEM((T, PW), jnp.int32),
            pltpu.VMEM((PW, 64), jnp.float32),
            pltpu.VMEM((PW, 64), jnp.float32),
            pltpu.VMEM_SHARED((NP, 64), jnp.float32),
            pltpu.SemaphoreType.DMA,
            pltpu.SemaphoreType.DMA,
        ],
        compiler_params=_SC_PARAMS,
    )(functools.partial(_prop_kernel, col_split=col_split))


_prop1_call = _make_prop(True)
_prop2_call = _make_prop(False)


_R = 2048
_GRID = (NP // _R,)


def _norms(degp_ref):
    deg_out = degp_ref[0, 0, :] + degp_ref[1, 0, :]
    deg_in = degp_ref[0, 1, :] + degp_ref[1, 1, :]
    no = lax.rsqrt(jnp.maximum(deg_out, 1.0))
    ni = lax.rsqrt(jnp.maximum(deg_in, 1.0))
    return no, ni


def _scale_kernel(x_ref, degp_ref, out_ref):
    no, _ = _norms(degp_ref)
    xn = x_ref[...] * no[:, None]
    out_ref[0] = xn[:, :64]
    out_ref[1] = xn[:, 64:]


def _mid_kernel(p_ref, degp_ref, w1_ref, b1_ref, w2_ref, out_ref):
    no, ni = _norms(degp_ref)
    agg = jnp.concatenate([p_ref[0], p_ref[1]], axis=1) * ni[:, None]
    h = jnp.maximum(jnp.dot(agg, w1_ref[...], preferred_element_type=jnp.float32)
                    + b1_ref[...], 0.0)
    out_ref[...] = jnp.dot(h, w2_ref[...],
                           preferred_element_type=jnp.float32) * no[:, None]


def _final_kernel(q_ref, degp_ref, b2_ref, out_ref):
    _, ni = _norms(degp_ref)
    out_ref[...] = (q_ref[0] + q_ref[1]) * ni[:, None] + b2_ref[...]


_DEGP_SPEC = pl.BlockSpec((NC, 2, _R), lambda i: (0, 0, i))


def _tc_scale(x, degp):
    return pl.pallas_call(
        _scale_kernel,
        grid=_GRID,
        in_specs=[pl.BlockSpec((_R, 128), lambda i: (i, 0)), _DEGP_SPEC],
        out_specs=pl.BlockSpec((NC, _R, 64), lambda i: (0, i, 0)),
        out_shape=jax.ShapeDtypeStruct((NC, NP, 64), jnp.float32),
    )(x, degp)


def _tc_mid(p, degp, W1, b1, W2):
    return pl.pallas_call(
        _mid_kernel,
        grid=_GRID,
        in_specs=[
            pl.BlockSpec((NC, _R, 64), lambda i: (0, i, 0)),
            _DEGP_SPEC,
            pl.BlockSpec((128, 128), lambda i: (0, 0)),
            pl.BlockSpec((1, 128), lambda i: (0, 0)),
            pl.BlockSpec((128, 64), lambda i: (0, 0)),
        ],
        out_specs=pl.BlockSpec((_R, 64), lambda i: (i, 0)),
        out_shape=jax.ShapeDtypeStruct((NP, 64), jnp.float32),
    )(p, degp, W1, b1, W2)


def _tc_final(q, degp, b2):
    return pl.pallas_call(
        _final_kernel,
        grid=_GRID,
        in_specs=[
            pl.BlockSpec((NC, _R, 64), lambda i: (0, i, 0)),
            _DEGP_SPEC,
            pl.BlockSpec((1, 64), lambda i: (0, 0)),
        ],
        out_specs=pl.BlockSpec((_R, 64), lambda i: (i, 0)),
        out_shape=jax.ShapeDtypeStruct((N_NODES, 64), jnp.float32),
    )(q, degp, b2)


def kernel(in_feat, edge_index, W1, b1, W2, b2):
    er = edge_index.astype(jnp.int32).reshape(2 * NW, N_EDGES // NW)
    npad = T * PW - N_EDGES // NW
    padv = N_NODES + (jnp.arange(npad, dtype=jnp.int32) % (NP - N_NODES))
    pads = jnp.broadcast_to(padv, (2 * NW, npad))
    ep = jnp.concatenate([er, pads], axis=1).reshape(2, NW, T, PW)
    srcx = jnp.stack([ep[0], ep[0] + NP])
    dstx = ep[1]

    degp = _degree_call(srcx, dstx).reshape(NC, 2, NP)
    xn1 = _tc_scale(in_feat, degp).reshape(NC * NP, 64)
    p1 = _prop1_call(xn1, srcx, dstx)
    g = _tc_mid(p1, degp, W1.astype(jnp.float32), b1.reshape(1, 128),
                W2.astype(jnp.float32))
    p2 = _prop2_call(g, srcx, dstx)
    return _tc_final(p2, degp, b2.reshape(1, 64))


# speedup vs baseline: 10.9422x; 1.0092x over previous
import functools

import jax
import jax.numpy as jnp
from jax import lax
from jax.experimental import pallas as pl
from jax.experimental.pallas import tpu as pltpu
from jax.experimental.pallas import tpu_sc as plsc

N_NODES = 10000
N_EDGES = 320000
NP = 10240

NC = 2
NS = 16
NW = NC * NS
PW = 128
T = 79
RPT = NP // NS
HPT = NP // NS

_SC_MESH = dict(core_axis_name="c", subcore_axis_name="s",
                num_cores=NC, num_subcores=NS)
_SC_PARAMS = pltpu.CompilerParams(use_tc_tiling_on_sc=False)


def _degree_kernel(srcx, dstx, degp, src_v, dst_v, ones_v, zb_v, hsrc, hdst):
    c = lax.axis_index("c")
    s = lax.axis_index("s")
    w = c * NS + s

    def _fill(i, _):
        zb_v[pl.ds(i * 16, 16)] = jnp.zeros((16,), jnp.float32)
        return 0
    lax.fori_loop(0, HPT // 16, _fill, 0)
    for i in range(PW // 16):
        ones_v[pl.ds(i * 16, 16)] = jnp.ones((16,), jnp.float32)

    pltpu.sync_copy(zb_v, hsrc.at[pl.ds(s * HPT, HPT)])
    pltpu.sync_copy(zb_v, hdst.at[pl.ds(s * HPT, HPT)])

    pltpu.sync_copy(srcx.at[w], src_v)
    pltpu.sync_copy(dstx.at[w], dst_v)
    plsc.subcore_barrier()

    def _body(j, _):
        pltpu.sync_copy(ones_v, hsrc.at[src_v.at[j]], add=True)
        pltpu.sync_copy(ones_v, hdst.at[dst_v.at[j]], add=True)
        return 0
    lax.fori_loop(0, T, _body, 0)

    plsc.subcore_barrier()
    pltpu.sync_copy(hsrc.at[pl.ds(s * HPT, HPT)],
                    degp.at[pl.ds((c * 2 + 0) * NP + s * HPT, HPT)])
    pltpu.sync_copy(hdst.at[pl.ds(s * HPT, HPT)],
                    degp.at[pl.ds((c * 2 + 1) * NP + s * HPT, HPT)])


_degree_call = functools.partial(
    pl.kernel,
    out_type=jax.ShapeDtypeStruct((NC * 2 * NP,), jnp.float32),
    mesh=plsc.VectorSubcoreMesh(**_SC_MESH),
    scratch_types=[
        pltpu.VMEM((T, PW), jnp.int32),
        pltpu.VMEM((T, PW), jnp.int32),
        pltpu.VMEM((PW,), jnp.float32),
        pltpu.VMEM((HPT,), jnp.float32),
        pltpu.VMEM_SHARED((NP,), jnp.float32),
        pltpu.VMEM_SHARED((NP,), jnp.float32),
    ],
    compiler_params=_SC_PARAMS,
)(_degree_kernel)


def _prop_kernel(feats, srcx, dstx, part, src_v, dst_v, rows0, rows1, acc,
                 sem0, sem1, *, col_split):
    c = lax.axis_index("c")
    s = lax.axis_index("s")

    def _fill(r, _):
        for i in range(4):
            rows0[r, pl.ds(i * 16, 16)] = jnp.zeros((16,), jnp.float32)
        return 0
    lax.fori_loop(0, PW, _fill, 0)
    base = s * RPT
    for k in range(RPT // PW):
        pltpu.sync_copy(rows0, acc.at[pl.ds(base + k * PW, PW)])

    def _wait(buf, sem):
        pltpu.make_async_copy(feats.at[pl.ds(0, PW)], buf, sem).wait()

    def _run(w, offset):
        pltpu.sync_copy(srcx.at[w], src_v)
        pltpu.sync_copy(dstx.at[w], dst_v)
        if offset:
            @pl.when(c == 1)
            def _():
                def _off(j, _):
                    for i in range(PW // 16):
                        src_v[j, pl.ds(i * 16, 16)] = (
                            src_v[j, pl.ds(i * 16, 16)] + NP)
                    return 0
                lax.fori_loop(0, T, _off, 0)
        plsc.subcore_barrier()
        pltpu.async_copy(feats.at[src_v.at[0]], rows0, sem0)

        def _body(i, _):
            j0 = 2 * i
            _wait(rows0, sem0)
            pltpu.async_copy(feats.at[src_v.at[j0 + 1]], rows1, sem1)
            pltpu.sync_copy(rows0, acc.at[dst_v.at[j0]], add=True)
            _wait(rows1, sem1)
            pltpu.async_copy(feats.at[src_v.at[j0 + 2]], rows0, sem0)
            pltpu.sync_copy(rows1, acc.at[dst_v.at[j0 + 1]], add=True)
            return 0
        lax.fori_loop(0, (T - 1) // 2, _body, 0)

        _wait(rows0, sem0)
        pltpu.sync_copy(rows0, acc.at[dst_v.at[T - 1]], add=True)

    if col_split:
        _run(s, True)
        _run(s + NS, True)
    else:
        _run(c * NS + s, False)

    plsc.subcore_barrier()
    pltpu.sync_copy(acc.at[pl.ds(base, RPT)], part.at[c, pl.ds(base, RPT)])


def _make_prop(col_split):
    return functools.partial(
        pl.kernel,
        out_type=jax.ShapeDtypeStruct((NC, NP, 64), jnp.float32),
        mesh=plsc.VectorSubcoreMesh(**_SC_MESH),
        scratch_types=[
            pltpu.VMEM((T, PW), jnp.int32),
            pltpu.VMEM((T, PW), jnp.int32),
            pltpu.VMEM((PW, 64), jnp.float32),
            pltpu.VMEM((PW, 64), jnp.float32),
            pltpu.VMEM_SHARED((NP, 64), jnp.float32),
            pltpu.SemaphoreType.DMA,
            pltpu.SemaphoreType.DMA,
        ],
        compiler_params=_SC_PARAMS,
    )(functools.partial(_prop_kernel, col_split=col_split))


_prop1_call = _make_prop(True)
_prop2_call = _make_prop(False)


_R = 2048
_GRID = (NP // _R,)


def _norms(degp_ref):
    deg_out = degp_ref[0, 0, :] + degp_ref[1, 0, :]
    deg_in = degp_ref[0, 1, :] + degp_ref[1, 1, :]
    no = lax.rsqrt(jnp.maximum(deg_out, 1.0))
    ni = lax.rsqrt(jnp.maximum(deg_in, 1.0))
    return no, ni


def _scale_kernel(x_ref, degp_ref, out_ref):
    no, _ = _norms(degp_ref)
    xn = x_ref[...] * no[:, None]
    out_ref[0] = xn[:, :64]
    out_ref[1] = xn[:, 64:]


def _mid_kernel(p_ref, degp_ref, w1_ref, b1_ref, w2_ref, out_ref):
    no, ni = _norms(degp_ref)
    agg = jnp.concatenate([p_ref[0], p_ref[1]], axis=1) * ni[:, None]
    h = jnp.maximum(jnp.dot(agg, w1_ref[...], preferred_element_type=jnp.float32)
                    + b1_ref[...], 0.0)
    out_ref[...] = jnp.dot(h, w2_ref[...],
                           preferred_element_type=jnp.float32) * no[:, None]


def _final_kernel(q_ref, degp_ref, b2_ref, out_ref):
    _, ni = _norms(degp_ref)
    out_ref[...] = (q_ref[0] + q_ref[1]) * ni[:, None] + b2_ref[...]


_DEGP_SPEC = pl.BlockSpec((NC, 2, _R), lambda i: (0, 0, i))


def _tc_scale(x, degp):
    return pl.pallas_call(
        _scale_kernel,
        grid=_GRID,
        in_specs=[pl.BlockSpec((_R, 128), lambda i: (i, 0)), _DEGP_SPEC],
        out_specs=pl.BlockSpec((NC, _R, 64), lambda i: (0, i, 0)),
        out_shape=jax.ShapeDtypeStruct((NC, NP, 64), jnp.float32),
    )(x, degp)


def _tc_mid(p, degp, W1, b1, W2):
    return pl.pallas_call(
        _mid_kernel,
        grid=_GRID,
        in_specs=[
            pl.BlockSpec((NC, _R, 64), lambda i: (0, i, 0)),
            _DEGP_SPEC,
            pl.BlockSpec((128, 128), lambda i: (0, 0)),
            pl.BlockSpec((1, 128), lambda i: (0, 0)),
            pl.BlockSpec((128, 64), lambda i: (0, 0)),
        ],
        out_specs=pl.BlockSpec((_R, 64), lambda i: (i, 0)),
        out_shape=jax.ShapeDtypeStruct((NP, 64), jnp.float32),
    )(p, degp, W1, b1, W2)


def _tc_final(q, degp, b2):
    return pl.pallas_call(
        _final_kernel,
        grid=_GRID,
        in_specs=[
            pl.BlockSpec((NC, _R, 64), lambda i: (0, i, 0)),
            _DEGP_SPEC,
            pl.BlockSpec((1, 64), lambda i: (0, 0)),
        ],
        out_specs=pl.BlockSpec((_R, 64), lambda i: (i, 0)),
        out_shape=jax.ShapeDtypeStruct((N_NODES, 64), jnp.float32),
    )(q, degp, b2)


def kernel(in_feat, edge_index, W1, b1, W2, b2):
    er = edge_index.astype(jnp.int32).reshape(2 * NW, N_EDGES // NW)
    npad = T * PW - N_EDGES // NW
    padv = N_NODES + (jnp.arange(npad, dtype=jnp.int32) % (NP - N_NODES))
    pads = jnp.broadcast_to(padv, (2 * NW, npad))
    ep = jnp.concatenate([er, pads], axis=1).reshape(2, NW, T, PW)
    srcx = ep[0]
    dstx = ep[1]

    degp = _degree_call(srcx, dstx).reshape(NC, 2, NP)
    xn1 = _tc_scale(in_feat, degp).reshape(NC * NP, 64)
    p1 = _prop1_call(xn1, srcx, dstx)
    g = _tc_mid(p1, degp, W1.astype(jnp.float32), b1.reshape(1, 128),
                W2.astype(jnp.float32))
    p2 = _prop2_call(g, srcx, dstx)
    return _tc_final(p2, degp, b2.reshape(1, 64))
